# Initial kernel scaffold; baseline (speedup 1.0000x reference)
import jax
import jax.numpy as jnp
from jax.experimental import pallas as pl


def kernel(xyz1, xyz2, points1, points2, W1, b1, W2, b2):
    raise NotImplementedError("write your pallas kernel here")


# speedup vs baseline: 2.0467x; 2.0467x over previous
import jax
import jax.numpy as jnp
from jax.experimental import pallas as pl


def _fused_body(xyz1_ref, xyz2t_ref, p1_ref, p2_ref, w1a_ref, w1b_ref,
                b1_ref, w2_ref, b2_ref, out_ref):
    x1 = xyz1_ref[0]
    x2 = xyz2t_ref[0]
    d = (x1[:, 0:1] - x2[0:1, :]) ** 2
    d = d + (x1[:, 1:2] - x2[1:2, :]) ** 2
    d = d + (x1[:, 2:3] - x2[2:3, :]) ** 2
    n2 = d.shape[1]
    dmin = jnp.min(d, axis=1, keepdims=True)
    jidx = jax.lax.broadcasted_iota(jnp.int32, d.shape, 1)
    idx = jnp.min(jnp.where(d == dmin, jidx, n2), axis=1, keepdims=True)
    onehot = (jidx == idx).astype(jnp.float32)
    interp = jax.lax.dot_general(
        onehot, p2_ref[0], (((1,), (0,)), ((), ())),
        precision=jax.lax.Precision.HIGHEST,
        preferred_element_type=jnp.float32)
    h = interp @ w1a_ref[...] + p1_ref[0] @ w1b_ref[...] + b1_ref[...]
    h = jnp.where(h >= 0, h, 0.2 * h)
    o = h @ w2_ref[...] + b2_ref[...]
    out_ref[0] = jnp.where(o >= 0, o, 0.2 * o)


def kernel(xyz1, xyz2, points1, points2, W1, b1, W2, b2):
    B, N1, _ = xyz1.shape
    N2 = xyz2.shape[1]
    F2 = points2.shape[2]
    FO = W2.shape[1]
    xyz2t = jnp.swapaxes(xyz2, 1, 2)
    W1a = W1[:F2]
    W1b = W1[F2:]
    b1r = b1.reshape(1, -1)
    b2r = b2.reshape(1, -1)

    RB = 512
    grid = (B, N1 // RB)
    return pl.pallas_call(
        _fused_body,
        grid=grid,
        in_specs=[
            pl.BlockSpec((1, RB, 3), lambda b, i: (b, i, 0)),
            pl.BlockSpec((1, 3, N2), lambda b, i: (b, 0, 0)),
            pl.BlockSpec((1, RB, points1.shape[2]), lambda b, i: (b, i, 0)),
            pl.BlockSpec((1, N2, F2), lambda b, i: (b, 0, 0)),
            pl.BlockSpec(W1a.shape, lambda b, i: (0, 0)),
            pl.BlockSpec(W1b.shape, lambda b, i: (0, 0)),
            pl.BlockSpec(b1r.shape, lambda b, i: (0, 0)),
            pl.BlockSpec(W2.shape, lambda b, i: (0, 0)),
            pl.BlockSpec(b2r.shape, lambda b, i: (0, 0)),
        ],
        out_specs=pl.BlockSpec((1, RB, FO), lambda b, i: (b, i, 0)),
        out_shape=jax.ShapeDtypeStruct((B, N1, FO), jnp.float32),
    )(xyz1, xyz2t, points1, points2, W1a, W1b, b1r, W2, b2r)


# speedup vs baseline: 3.1308x; 1.5297x over previous
import jax
import jax.numpy as jnp
from jax.experimental import pallas as pl


def _fused_body(xyz1_ref, xyz2t_ref, p1_ref, p2_ref, w1a_ref, w1b_ref,
                b1_ref, w2_ref, b2_ref, out_ref):
    x1 = xyz1_ref[0]
    x2 = xyz2t_ref[0]
    d = (x1[:, 0:1] - x2[0:1, :]) ** 2
    d = d + (x1[:, 1:2] - x2[1:2, :]) ** 2
    d = d + (x1[:, 2:3] - x2[2:3, :]) ** 2
    n2 = d.shape[1]
    dmin = jnp.min(d, axis=1, keepdims=True)
    jidx = jax.lax.broadcasted_iota(jnp.int32, d.shape, 1)
    idx = jnp.min(jnp.where(d == dmin, jidx, n2), axis=1, keepdims=True)
    onehot = (jidx == idx).astype(jnp.float32)
    interp = jax.lax.dot_general(
        onehot, p2_ref[0], (((1,), (0,)), ((), ())),
        precision=jax.lax.Precision.DEFAULT,
        preferred_element_type=jnp.float32)
    h = interp @ w1a_ref[...] + p1_ref[0] @ w1b_ref[...] + b1_ref[...]
    h = jnp.where(h >= 0, h, 0.2 * h)
    o = h @ w2_ref[...] + b2_ref[...]
    out_ref[0] = jnp.where(o >= 0, o, 0.2 * o)


def kernel(xyz1, xyz2, points1, points2, W1, b1, W2, b2):
    B, N1, _ = xyz1.shape
    N2 = xyz2.shape[1]
    F2 = points2.shape[2]
    FO = W2.shape[1]
    xyz2t = jnp.swapaxes(xyz2, 1, 2)
    W1a = W1[:F2]
    W1b = W1[F2:]
    b1r = b1.reshape(1, -1)
    b2r = b2.reshape(1, -1)

    RB = 512
    grid = (B, N1 // RB)
    return pl.pallas_call(
        _fused_body,
        grid=grid,
        in_specs=[
            pl.BlockSpec((1, RB, 3), lambda b, i: (b, i, 0)),
            pl.BlockSpec((1, 3, N2), lambda b, i: (b, 0, 0)),
            pl.BlockSpec((1, RB, points1.shape[2]), lambda b, i: (b, i, 0)),
            pl.BlockSpec((1, N2, F2), lambda b, i: (b, 0, 0)),
            pl.BlockSpec(W1a.shape, lambda b, i: (0, 0)),
            pl.BlockSpec(W1b.shape, lambda b, i: (0, 0)),
            pl.BlockSpec(b1r.shape, lambda b, i: (0, 0)),
            pl.BlockSpec(W2.shape, lambda b, i: (0, 0)),
            pl.BlockSpec(b2r.shape, lambda b, i: (0, 0)),
        ],
        out_specs=pl.BlockSpec((1, RB, FO), lambda b, i: (b, i, 0)),
        out_shape=jax.ShapeDtypeStruct((B, N1, FO), jnp.float32),
    )(xyz1, xyz2t, points1, points2, W1a, W1b, b1r, W2, b2r)
